# Initial kernel scaffold; baseline (speedup 1.0000x reference)
import jax
import jax.numpy as jnp
from jax.experimental import pallas as pl


def kernel(pred_class_logits, pred_bounding_boxes, gt_class, gt_bounding_boxes):
    raise NotImplementedError("write your pallas kernel here")


# speedup vs baseline: 1.9367x; 1.9367x over previous
import functools

import jax
import jax.numpy as jnp
from jax.experimental import pallas as pl
from jax.experimental.pallas import tpu as pltpu

_NP = 20000
_NG = 1000
_C = 256
_BP = 1000
_NB = _NP // _BP
_IOU_T = 0.3


def _loss_body(labels_ref, gt_ref, pbox_ref, logits_ref, out_ref,
               cnt_ref, pick_ref, lsem_ref, sl1_ref,
               fbmax_ref, fbce_ref, fbsl1_ref):
    i = pl.program_id(0)

    @pl.when(i == 0)
    def _init():
        cnt_ref[...] = jnp.zeros_like(cnt_ref)
        pick_ref[...] = jnp.zeros_like(pick_ref)
        lsem_ref[...] = jnp.zeros_like(lsem_ref)
        sl1_ref[...] = jnp.zeros_like(sl1_ref)
        fbmax_ref[...] = jnp.full_like(fbmax_ref, -1.0)
        fbce_ref[...] = jnp.zeros_like(fbce_ref)
        fbsl1_ref[...] = jnp.zeros_like(fbsl1_ref)

    x = logits_ref[...]
    rowmax = jnp.max(x, axis=1, keepdims=True)
    lse = rowmax + jnp.log(jnp.sum(jnp.exp(x - rowmax), axis=1, keepdims=True))

    px1 = pbox_ref[:, 0:1]
    py1 = pbox_ref[:, 1:2]
    px2 = pbox_ref[:, 2:3]
    py2 = pbox_ref[:, 3:4]
    gx1 = gt_ref[0:1, :]
    gy1 = gt_ref[1:2, :]
    gx2 = gt_ref[2:3, :]
    gy2 = gt_ref[3:4, :]

    area_p = (px2 - px1) * (py2 - py1)
    area_g = (gx2 - gx1) * (gy2 - gy1)
    wx = jnp.maximum(jnp.minimum(px2, gx2) - jnp.maximum(px1, gx1), 0.0)
    wy = jnp.maximum(jnp.minimum(py2, gy2) - jnp.maximum(py1, gy1), 0.0)
    inter = wx * wy
    iou = inter / (area_p + area_g - inter)
    mask = (iou > _IOU_T).astype(jnp.float32)

    lab = labels_ref[0:1, :]
    onehot = (jax.lax.broadcasted_iota(jnp.int32, (_C, _NG), 0) == lab
              ).astype(jnp.bfloat16)
    p_mat = jax.lax.dot_general(
        x.astype(jnp.bfloat16), onehot,
        dimension_numbers=(((1,), (0,)), ((), ())),
        preferred_element_type=jnp.float32)

    cnt_ref[...] += jnp.sum(mask, keepdims=True)
    pick_ref[...] += jnp.sum(mask * p_mat, keepdims=True)
    rowcnt = jnp.sum(mask, axis=1, keepdims=True)
    lsem_ref[...] += jnp.sum(rowcnt * lse, keepdims=True)

    s = jnp.zeros((1, 1), jnp.float32)
    for pk, gk in ((px1, gx1), (py1, gy1), (px2, gx2), (py2, gy2)):
        d = pk - gk
        ad = jnp.abs(d)
        s = s + jnp.where(ad < 1.0, 0.5 * d * d, ad - 0.5)
    sl1_ref[...] += jnp.sum(mask * s, keepdims=True)

    bmax = jnp.max(iou, axis=0, keepdims=True)
    ridx = jax.lax.broadcasted_iota(jnp.int32, (_BP, _NG), 0)
    cand_rows = jnp.where(iou == bmax, ridx, _BP)
    minidx = jnp.min(cand_rows, axis=0, keepdims=True)
    sel = (ridx == minidx).astype(jnp.float32)
    cand_ce = jnp.sum(sel * (lse - p_mat), axis=0, keepdims=True)
    cand_sl1 = jnp.sum(sel * s, axis=0, keepdims=True)
    upd = bmax > fbmax_ref[...]
    fbce_ref[...] = jnp.where(upd, cand_ce, fbce_ref[...])
    fbsl1_ref[...] = jnp.where(upd, cand_sl1, fbsl1_ref[...])
    fbmax_ref[...] = jnp.where(upd, bmax, fbmax_ref[...])

    @pl.when(i == _NB - 1)
    def _finalize():
        count = cnt_ref[...]
        main = ((lsem_ref[...] - pick_ref[...]) / count
                + sl1_ref[...] / (4.0 * count))
        keep = (fbmax_ref[...] > 0.0).astype(jnp.float32)
        dfb = jnp.sum(keep, keepdims=True)
        fb = (jnp.sum(keep * fbce_ref[...], keepdims=True) / dfb
              + jnp.sum(keep * fbsl1_ref[...], keepdims=True) / (4.0 * dfb))
        out_ref[...] = jnp.where(count > 0.0, main, fb)


@functools.partial(jax.jit, static_argnames=())
def kernel(pred_class_logits, pred_bounding_boxes, gt_class, gt_bounding_boxes):
    labels = jnp.broadcast_to(
        gt_class[0].astype(jnp.int32)[None, :], (8, _NG))
    gt_t = jnp.zeros((8, _NG), jnp.float32).at[:4].set(gt_bounding_boxes[0].T)

    out = pl.pallas_call(
        _loss_body,
        grid=(_NB,),
        in_specs=[
            pl.BlockSpec((8, _NG), lambda i: (0, 0)),
            pl.BlockSpec((8, _NG), lambda i: (0, 0)),
            pl.BlockSpec((_BP, 4), lambda i: (i, 0)),
            pl.BlockSpec((_BP, _C), lambda i: (i, 0)),
        ],
        out_specs=pl.BlockSpec((1, 1), lambda i: (0, 0)),
        out_shape=jax.ShapeDtypeStruct((1, 1), jnp.float32),
        scratch_shapes=[
            pltpu.VMEM((1, 1), jnp.float32),
            pltpu.VMEM((1, 1), jnp.float32),
            pltpu.VMEM((1, 1), jnp.float32),
            pltpu.VMEM((1, 1), jnp.float32),
            pltpu.VMEM((1, _NG), jnp.float32),
            pltpu.VMEM((1, _NG), jnp.float32),
            pltpu.VMEM((1, _NG), jnp.float32),
        ],
    )(labels, gt_t, pred_bounding_boxes, pred_class_logits)
    return out[0, 0]


# speedup vs baseline: 2.1854x; 1.1284x over previous
import functools

import jax
import jax.numpy as jnp
from jax.experimental import pallas as pl
from jax.experimental.pallas import tpu as pltpu

_NP = 20000
_NG = 1000
_C = 256
_BP = 1000
_NB = _NP // _BP
_IOU_T = 0.3


def _loss_body(labels_ref, gt_ref, pbox_ref, logits_ref, out_ref,
               cnt_ref, pick_ref, lsem_ref, sl1_ref,
               fbmax_ref, fbce_ref):
    i = pl.program_id(0)

    @pl.when(i == 0)
    def _init():
        cnt_ref[...] = jnp.zeros_like(cnt_ref)
        pick_ref[...] = jnp.zeros_like(pick_ref)
        lsem_ref[...] = jnp.zeros_like(lsem_ref)
        sl1_ref[...] = jnp.zeros_like(sl1_ref)
        fbmax_ref[...] = jnp.full_like(fbmax_ref, -1.0)
        fbce_ref[...] = jnp.zeros_like(fbce_ref)

    x = logits_ref[...]
    rowmax = jnp.max(x, axis=1, keepdims=True)
    lse = rowmax + jnp.log(jnp.sum(jnp.exp(x - rowmax), axis=1, keepdims=True))

    px1 = pbox_ref[:, 0:1]
    py1 = pbox_ref[:, 1:2]
    px2 = pbox_ref[:, 2:3]
    py2 = pbox_ref[:, 3:4]
    gx1 = gt_ref[0:1, :]
    gy1 = gt_ref[1:2, :]
    gx2 = gt_ref[2:3, :]
    gy2 = gt_ref[3:4, :]

    area_p = (px2 - px1) * (py2 - py1)
    area_g = (gx2 - gx1) * (gy2 - gy1)
    wx = jnp.maximum(jnp.minimum(px2, gx2) - jnp.maximum(px1, gx1), 0.0)
    wy = jnp.maximum(jnp.minimum(py2, gy2) - jnp.maximum(py1, gy1), 0.0)
    inter = wx * wy
    iou = inter / (area_p + area_g - inter)
    mask = (iou > _IOU_T).astype(jnp.float32)

    lab = labels_ref[0:1, :]
    onehot = (jax.lax.broadcasted_iota(jnp.int32, (_C, _NG), 0) == lab
              ).astype(jnp.bfloat16)
    p_mat = jax.lax.dot_general(
        x.astype(jnp.bfloat16), onehot,
        dimension_numbers=(((1,), (0,)), ((), ())),
        preferred_element_type=jnp.float32)

    rowcnt = jnp.sum(mask, axis=1, keepdims=True)
    cnt_ref[...] += jnp.sum(rowcnt, keepdims=True)
    pick_ref[...] += jnp.sum(mask * p_mat, keepdims=True)
    lsem_ref[...] += jnp.sum(rowcnt * lse, keepdims=True)

    s_raw = None
    for pk, gk in ((px1, gx1), (py1, gy1), (px2, gx2), (py2, gy2)):
        ad = jnp.abs(pk - gk)
        m = jnp.minimum(ad, 1.0)
        t = m * (ad + ad - m)
        s_raw = t if s_raw is None else s_raw + t
    sl1_ref[...] += 0.5 * jnp.sum(mask * s_raw, keepdims=True)

    bmax = jnp.max(iou, axis=0, keepdims=True)
    ridx = jax.lax.broadcasted_iota(jnp.int32, (_BP, _NG), 0)
    cand_rows = jnp.where(iou == bmax, ridx, _BP)
    minidx = jnp.min(cand_rows, axis=0, keepdims=True)
    sel = (ridx == minidx).astype(jnp.float32)
    cand = jnp.sum(sel * ((lse - p_mat) + 0.125 * s_raw),
                   axis=0, keepdims=True)
    upd = bmax > fbmax_ref[...]
    fbce_ref[...] = jnp.where(upd, cand, fbce_ref[...])
    fbmax_ref[...] = jnp.where(upd, bmax, fbmax_ref[...])

    @pl.when(i == _NB - 1)
    def _finalize():
        count = cnt_ref[...]
        main = ((lsem_ref[...] - pick_ref[...]) / count
                + sl1_ref[...] / (4.0 * count))
        keep = (fbmax_ref[...] > 0.0).astype(jnp.float32)
        dfb = jnp.sum(keep, keepdims=True)
        fb = jnp.sum(keep * fbce_ref[...], keepdims=True) / dfb
        out_ref[...] = jnp.where(count > 0.0, main, fb)


@functools.partial(jax.jit, static_argnames=())
def kernel(pred_class_logits, pred_bounding_boxes, gt_class, gt_bounding_boxes):
    labels = jnp.broadcast_to(
        gt_class[0].astype(jnp.int32)[None, :], (8, _NG))
    gt_t = jnp.zeros((8, _NG), jnp.float32).at[:4].set(gt_bounding_boxes[0].T)

    out = pl.pallas_call(
        _loss_body,
        grid=(_NB,),
        in_specs=[
            pl.BlockSpec((8, _NG), lambda i: (0, 0)),
            pl.BlockSpec((8, _NG), lambda i: (0, 0)),
            pl.BlockSpec((_BP, 4), lambda i: (i, 0)),
            pl.BlockSpec((_BP, _C), lambda i: (i, 0)),
        ],
        out_specs=pl.BlockSpec((1, 1), lambda i: (0, 0)),
        out_shape=jax.ShapeDtypeStruct((1, 1), jnp.float32),
        scratch_shapes=[
            pltpu.VMEM((1, 1), jnp.float32),
            pltpu.VMEM((1, 1), jnp.float32),
            pltpu.VMEM((1, 1), jnp.float32),
            pltpu.VMEM((1, 1), jnp.float32),
            pltpu.VMEM((1, _NG), jnp.float32),
            pltpu.VMEM((1, _NG), jnp.float32),
        ],
    )(labels, gt_t, pred_bounding_boxes, pred_class_logits)
    return out[0, 0]
